# Initial kernel scaffold; baseline (speedup 1.0000x reference)
import jax
import jax.numpy as jnp
from jax.experimental import pallas as pl


def kernel(x, edge_index, W1, b1, g1, bt1, W2, b2, g2, bt2, W3, b3, g3, bt3, fcW, fcb):
    raise NotImplementedError("write your pallas kernel here")


# speedup vs baseline: 10.4569x; 10.4569x over previous
import functools

import jax
import jax.numpy as jnp
from jax import lax
from jax.experimental import pallas as pl
from jax.experimental.pallas import tpu as pltpu
from jax.experimental.pallas import tpu_sc as plsc

N = 10000
E = 320000
EPS = 1e-5
_S = (1.0 + EPS) ** -0.5

NC = 2
NS = 16
NW = NC * NS
CHUNK = 128
NCHUNKS = 80
EPW = NCHUNKS * CHUNK
E_PAD = EPW * NW
N_ACC = 10112
RPT = N_ACC // NS
DEG_W = 16

_mesh = plsc.VectorSubcoreMesh(
    core_axis_name="c", subcore_axis_name="s", num_cores=NC, num_subcores=NS
)
_sc_params = pltpu.CompilerParams(use_tc_tiling_on_sc=False)


@functools.partial(
    pl.kernel,
    out_type=jax.ShapeDtypeStruct((NC, N_ACC, DEG_W), jnp.float32),
    mesh=_mesh,
    compiler_params=_sc_params,
    scratch_types=[
        pltpu.VMEM((NCHUNKS, CHUNK), jnp.int32),
        pltpu.VMEM((CHUNK, DEG_W), jnp.float32),
        pltpu.VMEM_SHARED((N_ACC, DEG_W), jnp.float32),
    ],
)
def _deg_kernel(dst_hbm, zeros_hbm, out_hbm, dstv, ones_v, acc):
    cid = lax.axis_index("c")
    sid = lax.axis_index("s")
    wid = sid * NC + cid
    pltpu.sync_copy(zeros_hbm.at[pl.ds(sid * RPT, RPT)],
                    acc.at[pl.ds(sid * RPT, RPT)])
    pltpu.sync_copy(dst_hbm.at[wid], dstv)

    def fill(i, carry):
        ones_v[i, :] = jnp.ones((16,), jnp.float32)
        return carry

    lax.fori_loop(0, CHUNK, fill, 0)
    plsc.subcore_barrier()

    def body(j, carry):
        pltpu.sync_copy(ones_v, acc.at[dstv.at[j]], add=True)
        return carry

    lax.fori_loop(0, NCHUNKS, body, 0)
    plsc.subcore_barrier()
    pltpu.sync_copy(acc.at[pl.ds(sid * RPT, RPT)],
                    out_hbm.at[cid, pl.ds(sid * RPT, RPT)])


def _make_scatter(W):
    @functools.partial(
        pl.kernel,
        out_type=jax.ShapeDtypeStruct((NC, N_ACC, W), jnp.float32),
        mesh=_mesh,
        compiler_params=_sc_params,
        scratch_types=[
            pltpu.VMEM((NCHUNKS, CHUNK), jnp.int32),
            pltpu.VMEM((NCHUNKS, CHUNK), jnp.int32),
            pltpu.VMEM((CHUNK, W), jnp.float32),
            pltpu.VMEM_SHARED((N_ACC, W), jnp.float32),
            pltpu.SemaphoreType.DMA,
        ],
    )
    def _scatter(xs_hbm, src_hbm, dst_hbm, zeros_hbm, out_hbm,
                 srcv, dstv, rows, acc, sem):
        cid = lax.axis_index("c")
        sid = lax.axis_index("s")
        wid = sid * NC + cid
        pltpu.sync_copy(zeros_hbm.at[pl.ds(sid * RPT, RPT)],
                        acc.at[pl.ds(sid * RPT, RPT)])
        pltpu.sync_copy(src_hbm.at[wid], srcv)
        pltpu.sync_copy(dst_hbm.at[wid], dstv)
        plsc.subcore_barrier()

        def body(j, carry):
            pltpu.async_copy(xs_hbm.at[srcv.at[j]], rows, sem).wait()
            pltpu.sync_copy(rows, acc.at[dstv.at[j]], add=True)
            return carry

        lax.fori_loop(0, NCHUNKS, body, 0)
        plsc.subcore_barrier()
        pltpu.sync_copy(acc.at[pl.ds(sid * RPT, RPT)],
                        out_hbm.at[cid, pl.ds(sid * RPT, RPT)])

    return _scatter


_scatter128 = _make_scatter(128)
_scatter64 = _make_scatter(64)
_scatter32 = _make_scatter(32)


R = 1000
G = N // R


def _row_spec(w):
    return pl.BlockSpec((R, w), lambda i: (i, 0))


def _full_spec(shape):
    return pl.BlockSpec(shape, lambda i: tuple(0 for _ in shape))


def _tca_body(p0, p1, x, dinv_o, x1s_o):
    deg = 1.0 + p0[:, 0:1] + p1[:, 0:1]
    dinv = lax.rsqrt(deg)
    dinv_o[...] = dinv
    x1s_o[...] = x[...] * dinv


_tc_a = pl.pallas_call(
    _tca_body,
    grid=(G,),
    in_specs=[_row_spec(DEG_W), _row_spec(DEG_W), _row_spec(128)],
    out_specs=[_row_spec(1), _row_spec(128)],
    out_shape=[
        jax.ShapeDtypeStruct((N, 1), jnp.float32),
        jax.ShapeDtypeStruct((N, 128), jnp.float32),
    ],
)


def _tcb_body(dinv, p1a, p1b, x1s, W1r, g1r, b1r, bt1r, W2r, g2r, x2s_o):
    di = dinv[...]
    g1s = g1r[...] * _S
    W1p = W1r[...] * g1s
    b1p = b1r[...] * g1s + bt1r[...]
    agg = (p1a[...] + p1b[...] + x1s[...]) * di
    h1 = jnp.maximum(
        jnp.dot(agg, W1p, preferred_element_type=jnp.float32) + b1p, 0.0)
    W2p = W2r[...] * (g2r[...] * _S)
    x2s_o[...] = jnp.dot(h1, W2p, preferred_element_type=jnp.float32) * di


_tc_b = pl.pallas_call(
    _tcb_body,
    grid=(G,),
    in_specs=[
        _row_spec(1), _row_spec(128), _row_spec(128), _row_spec(128),
        _full_spec((128, 128)), _full_spec((1, 128)), _full_spec((1, 128)),
        _full_spec((1, 128)), _full_spec((128, 64)), _full_spec((1, 64)),
    ],
    out_specs=_row_spec(64),
    out_shape=jax.ShapeDtypeStruct((N, 64), jnp.float32),
)


def _tcc_body(dinv, p2a, p2b, x2s, b2r, g2r, bt2r, W3r, g3r, x3s_o):
    di = dinv[...]
    g2s = g2r[...] * _S
    b2p = b2r[...] * g2s + bt2r[...]
    h2 = jnp.maximum((p2a[...] + p2b[...] + x2s[...]) * di + b2p, 0.0)
    W3p = W3r[...] * (g3r[...] * _S)
    x3s_o[...] = jnp.dot(h2, W3p, preferred_element_type=jnp.float32) * di


_tc_c = pl.pallas_call(
    _tcc_body,
    grid=(G,),
    in_specs=[
        _row_spec(1), _row_spec(64), _row_spec(64), _row_spec(64),
        _full_spec((1, 64)), _full_spec((1, 64)), _full_spec((1, 64)),
        _full_spec((64, 32)), _full_spec((1, 32)),
    ],
    out_specs=_row_spec(32),
    out_shape=jax.ShapeDtypeStruct((N, 32), jnp.float32),
)


def _tcd_body(dinv, p3a, p3b, x3s, b3r, g3r, bt3r, fcWr, fcbr, out_o):
    di = dinv[...]
    g3s = g3r[...] * _S
    b3p = b3r[...] * g3s + bt3r[...]
    h3 = jnp.maximum((p3a[...] + p3b[...] + x3s[...]) * di + b3p, 0.0)
    out_o[...] = (jnp.dot(h3, fcWr[...], preferred_element_type=jnp.float32)
                  + fcbr[...])


_tc_d = pl.pallas_call(
    _tcd_body,
    grid=(G,),
    in_specs=[
        _row_spec(1), _row_spec(32), _row_spec(32), _row_spec(32),
        _full_spec((1, 32)), _full_spec((1, 32)), _full_spec((1, 32)),
        _full_spec((32, 1)), _full_spec((1, 1)),
    ],
    out_specs=_row_spec(1),
    out_shape=jax.ShapeDtypeStruct((N, 1), jnp.float32),
)


def kernel(x, edge_index, W1, b1, g1, bt1, W2, b2, g2, bt2,
           W3, b3, g3, bt3, fcW, fcb):
    src = edge_index[0].astype(jnp.int32)
    dst = edge_index[1].astype(jnp.int32)
    npad = E_PAD - E
    src3 = jnp.concatenate(
        [src, jnp.zeros((npad,), jnp.int32)]).reshape(NW, NCHUNKS, CHUNK)
    dst3 = jnp.concatenate(
        [dst, jnp.full((npad,), N, jnp.int32)]).reshape(NW, NCHUNKS, CHUNK)
    z16 = jnp.zeros((N_ACC, DEG_W), jnp.float32)
    z128 = jnp.zeros((N_ACC, 128), jnp.float32)
    z64 = jnp.zeros((N_ACC, 64), jnp.float32)
    z32 = jnp.zeros((N_ACC, 32), jnp.float32)

    degP = _deg_kernel(dst3, z16)
    dinv, x1s = _tc_a(degP[0, :N], degP[1, :N], x)
    p1 = _scatter128(x1s, src3, dst3, z128)
    x2s = _tc_b(dinv, p1[0, :N], p1[1, :N], x1s, W1,
                g1.reshape(1, 128), b1.reshape(1, 128), bt1.reshape(1, 128),
                W2, g2.reshape(1, 64))
    p2 = _scatter64(x2s, src3, dst3, z64)
    x3s = _tc_c(dinv, p2[0, :N], p2[1, :N], x2s,
                b2.reshape(1, 64), g2.reshape(1, 64), bt2.reshape(1, 64),
                W3, g3.reshape(1, 32))
    p3 = _scatter32(x3s, src3, dst3, z32)
    out = _tc_d(dinv, p3[0, :N], p3[1, :N], x3s,
                b3.reshape(1, 32), g3.reshape(1, 32), bt3.reshape(1, 32),
                fcW, fcb.reshape(1, 1))
    return out


# speedup vs baseline: 11.9174x; 1.1397x over previous
import functools

import jax
import jax.numpy as jnp
from jax import lax
from jax.experimental import pallas as pl
from jax.experimental.pallas import tpu as pltpu
from jax.experimental.pallas import tpu_sc as plsc

N = 10000
E = 320000
EPS = 1e-5
_S = (1.0 + EPS) ** -0.5

NC = 2
NS = 16
NW = NC * NS
CHUNK = 128
NCHUNKS = 80
EPW = NCHUNKS * CHUNK
E_PAD = EPW * NW
N_ACC = 10112
RPT = N_ACC // NS
DEG_W = 16

_mesh = plsc.VectorSubcoreMesh(
    core_axis_name="c", subcore_axis_name="s", num_cores=NC, num_subcores=NS
)
_sc_params = pltpu.CompilerParams(use_tc_tiling_on_sc=False)


@functools.partial(
    pl.kernel,
    out_type=jax.ShapeDtypeStruct((NC, N_ACC, DEG_W), jnp.float32),
    mesh=_mesh,
    compiler_params=_sc_params,
    scratch_types=[
        pltpu.VMEM((NCHUNKS, CHUNK), jnp.int32),
        pltpu.VMEM((CHUNK, DEG_W), jnp.float32),
        pltpu.VMEM_SHARED((N_ACC, DEG_W), jnp.float32),
    ],
)
def _deg_kernel(dst_hbm, zeros_hbm, out_hbm, dstv, ones_v, acc):
    cid = lax.axis_index("c")
    sid = lax.axis_index("s")
    wid = sid * NC + cid
    pltpu.sync_copy(zeros_hbm.at[pl.ds(sid * RPT, RPT)],
                    acc.at[pl.ds(sid * RPT, RPT)])
    pltpu.sync_copy(dst_hbm.at[wid], dstv)

    def fill(i, carry):
        ones_v[i, :] = jnp.ones((16,), jnp.float32)
        return carry

    lax.fori_loop(0, CHUNK, fill, 0)
    plsc.subcore_barrier()

    def body(j, carry):
        pltpu.sync_copy(ones_v, acc.at[dstv.at[j]], add=True)
        return carry

    lax.fori_loop(0, NCHUNKS, body, 0)
    plsc.subcore_barrier()
    pltpu.sync_copy(acc.at[pl.ds(sid * RPT, RPT)],
                    out_hbm.at[cid, pl.ds(sid * RPT, RPT)])


def _make_scatter(W, nbuf, nphase):
    pchunks = NCHUNKS // nphase
    ng = pchunks // nbuf

    @functools.partial(
        pl.kernel,
        out_type=jax.ShapeDtypeStruct((NC, N_ACC, W), jnp.float32),
        mesh=_mesh,
        compiler_params=_sc_params,
        scratch_types=[
            pltpu.VMEM((pchunks, CHUNK), jnp.int32),
            pltpu.VMEM((pchunks, CHUNK), jnp.int32),
            [pltpu.VMEM((CHUNK, W), jnp.float32) for _ in range(nbuf)],
            pltpu.VMEM_SHARED((N_ACC, W), jnp.float32),
            [pltpu.SemaphoreType.DMA for _ in range(nbuf)],
        ],
    )
    def _scatter(xs_hbm, src_hbm, dst_hbm, zeros_hbm, out_hbm,
                 srcv, dstv, rows, acc, sems):
        cid = lax.axis_index("c")
        sid = lax.axis_index("s")
        wid = sid * NC + cid
        pltpu.sync_copy(zeros_hbm.at[pl.ds(sid * RPT, RPT)],
                        acc.at[pl.ds(sid * RPT, RPT)])
        plsc.subcore_barrier()

        for p in range(nphase):
            pltpu.sync_copy(src_hbm.at[wid, pl.ds(p * pchunks, pchunks)],
                            srcv)
            pltpu.sync_copy(dst_hbm.at[wid, pl.ds(p * pchunks, pchunks)],
                            dstv)

            for b in range(nbuf):
                pltpu.async_copy(xs_hbm.at[srcv.at[b]], rows[b], sems[b])

            def body(g, carry):
                jbase = g * nbuf
                for b in range(nbuf):
                    j = jbase + b
                    pltpu.make_async_copy(
                        xs_hbm.at[srcv.at[j]], rows[b], sems[b]).wait()
                    pltpu.sync_copy(rows[b], acc.at[dstv.at[j]], add=True)
                    nj = j + nbuf

                    @pl.when(nj < pchunks)
                    def _():
                        pltpu.async_copy(
                            xs_hbm.at[srcv.at[nj]], rows[b], sems[b])

                return carry

            lax.fori_loop(0, ng, body, 0)

        plsc.subcore_barrier()
        pltpu.sync_copy(acc.at[pl.ds(sid * RPT, RPT)],
                        out_hbm.at[cid, pl.ds(sid * RPT, RPT)])

    return _scatter


_scatter128 = _make_scatter(128, 2, 2)
_scatter64 = _make_scatter(64, 4, 1)
_scatter32 = _make_scatter(32, 4, 1)


R = 1000
G = N // R


def _row_spec(w):
    return pl.BlockSpec((R, w), lambda i: (i, 0))


def _full_spec(shape):
    return pl.BlockSpec(shape, lambda i: tuple(0 for _ in shape))


def _tca_body(p0, p1, x, dinv_o, x1s_o):
    deg = 1.0 + p0[:, 0:1] + p1[:, 0:1]
    dinv = lax.rsqrt(deg)
    dinv_o[...] = dinv
    x1s_o[...] = x[...] * dinv


_tc_a = pl.pallas_call(
    _tca_body,
    grid=(G,),
    in_specs=[_row_spec(DEG_W), _row_spec(DEG_W), _row_spec(128)],
    out_specs=[_row_spec(1), _row_spec(128)],
    out_shape=[
        jax.ShapeDtypeStruct((N, 1), jnp.float32),
        jax.ShapeDtypeStruct((N, 128), jnp.float32),
    ],
)


def _tcb_body(dinv, p1a, p1b, x1s, W1r, g1r, b1r, bt1r, W2r, g2r, x2s_o):
    di = dinv[...]
    g1s = g1r[...] * _S
    W1p = W1r[...] * g1s
    b1p = b1r[...] * g1s + bt1r[...]
    agg = (p1a[...] + p1b[...] + x1s[...]) * di
    h1 = jnp.maximum(
        jnp.dot(agg, W1p, preferred_element_type=jnp.float32) + b1p, 0.0)
    W2p = W2r[...] * (g2r[...] * _S)
    x2s_o[...] = jnp.dot(h1, W2p, preferred_element_type=jnp.float32) * di


_tc_b = pl.pallas_call(
    _tcb_body,
    grid=(G,),
    in_specs=[
        _row_spec(1), _row_spec(128), _row_spec(128), _row_spec(128),
        _full_spec((128, 128)), _full_spec((1, 128)), _full_spec((1, 128)),
        _full_spec((1, 128)), _full_spec((128, 64)), _full_spec((1, 64)),
    ],
    out_specs=_row_spec(64),
    out_shape=jax.ShapeDtypeStruct((N, 64), jnp.float32),
)


def _tcc_body(dinv, p2a, p2b, x2s, b2r, g2r, bt2r, W3r, g3r, x3s_o):
    di = dinv[...]
    g2s = g2r[...] * _S
    b2p = b2r[...] * g2s + bt2r[...]
    h2 = jnp.maximum((p2a[...] + p2b[...] + x2s[...]) * di + b2p, 0.0)
    W3p = W3r[...] * (g3r[...] * _S)
    x3s_o[...] = jnp.dot(h2, W3p, preferred_element_type=jnp.float32) * di


_tc_c = pl.pallas_call(
    _tcc_body,
    grid=(G,),
    in_specs=[
        _row_spec(1), _row_spec(64), _row_spec(64), _row_spec(64),
        _full_spec((1, 64)), _full_spec((1, 64)), _full_spec((1, 64)),
        _full_spec((64, 32)), _full_spec((1, 32)),
    ],
    out_specs=_row_spec(32),
    out_shape=jax.ShapeDtypeStruct((N, 32), jnp.float32),
)


def _tcd_body(dinv, p3a, p3b, x3s, b3r, g3r, bt3r, fcWr, fcbr, out_o):
    di = dinv[...]
    g3s = g3r[...] * _S
    b3p = b3r[...] * g3s + bt3r[...]
    h3 = jnp.maximum((p3a[...] + p3b[...] + x3s[...]) * di + b3p, 0.0)
    out_o[...] = (jnp.dot(h3, fcWr[...], preferred_element_type=jnp.float32)
                  + fcbr[...])


_tc_d = pl.pallas_call(
    _tcd_body,
    grid=(G,),
    in_specs=[
        _row_spec(1), _row_spec(32), _row_spec(32), _row_spec(32),
        _full_spec((1, 32)), _full_spec((1, 32)), _full_spec((1, 32)),
        _full_spec((32, 1)), _full_spec((1, 1)),
    ],
    out_specs=_row_spec(1),
    out_shape=jax.ShapeDtypeStruct((N, 1), jnp.float32),
)


def kernel(x, edge_index, W1, b1, g1, bt1, W2, b2, g2, bt2,
           W3, b3, g3, bt3, fcW, fcb):
    src = edge_index[0].astype(jnp.int32)
    dst = edge_index[1].astype(jnp.int32)
    npad = E_PAD - E
    src3 = jnp.concatenate(
        [src, jnp.zeros((npad,), jnp.int32)]).reshape(NW, NCHUNKS, CHUNK)
    dst3 = jnp.concatenate(
        [dst, jnp.full((npad,), N, jnp.int32)]).reshape(NW, NCHUNKS, CHUNK)
    z16 = jnp.zeros((N_ACC, DEG_W), jnp.float32)
    z128 = jnp.zeros((N_ACC, 128), jnp.float32)
    z64 = jnp.zeros((N_ACC, 64), jnp.float32)
    z32 = jnp.zeros((N_ACC, 32), jnp.float32)

    degP = _deg_kernel(dst3, z16)
    dinv, x1s = _tc_a(degP[0, :N], degP[1, :N], x)
    p1 = _scatter128(x1s, src3, dst3, z128)
    x2s = _tc_b(dinv, p1[0, :N], p1[1, :N], x1s, W1,
                g1.reshape(1, 128), b1.reshape(1, 128), bt1.reshape(1, 128),
                W2, g2.reshape(1, 64))
    p2 = _scatter64(x2s, src3, dst3, z64)
    x3s = _tc_c(dinv, p2[0, :N], p2[1, :N], x2s,
                b2.reshape(1, 64), g2.reshape(1, 64), bt2.reshape(1, 64),
                W3, g3.reshape(1, 32))
    p3 = _scatter32(x3s, src3, dst3, z32)
    out = _tc_d(dinv, p3[0, :N], p3[1, :N], x3s,
                b3.reshape(1, 32), g3.reshape(1, 32), bt3.reshape(1, 32),
                fcW, fcb.reshape(1, 1))
    return out


# speedup vs baseline: 17.2031x; 1.4435x over previous
import functools

import jax
import jax.numpy as jnp
from jax import lax
from jax.experimental import pallas as pl
from jax.experimental.pallas import tpu as pltpu
from jax.experimental.pallas import tpu_sc as plsc

N = 10000
E = 320000
EPS = 1e-5
_S = (1.0 + EPS) ** -0.5

NC = 2
NS = 16
NW = NC * NS
CHUNK = 128
NCHUNKS = 80
EPW = NCHUNKS * CHUNK
E_PAD = EPW * NW
N_ACC = 10112
RPT = N_ACC // NS
DEG_W = 16

_mesh = plsc.VectorSubcoreMesh(
    core_axis_name="c", subcore_axis_name="s", num_cores=NC, num_subcores=NS
)
_sc_params = pltpu.CompilerParams(use_tc_tiling_on_sc=False)
_sc_params_nl = pltpu.CompilerParams(use_tc_tiling_on_sc=False,
                                     needs_layout_passes=False)


@functools.partial(
    pl.kernel,
    out_type=jax.ShapeDtypeStruct((NC, N_ACC, DEG_W), jnp.float32),
    mesh=_mesh,
    compiler_params=_sc_params,
    scratch_types=[
        pltpu.VMEM((NCHUNKS, CHUNK), jnp.int32),
        pltpu.VMEM((CHUNK, DEG_W), jnp.float32),
        pltpu.VMEM_SHARED((N_ACC, DEG_W), jnp.float32),
    ],
)
def _deg_kernel(dst_hbm, zeros_hbm, out_hbm, dstv, ones_v, acc):
    cid = lax.axis_index("c")
    sid = lax.axis_index("s")
    wid = sid * NC + cid
    pltpu.sync_copy(zeros_hbm.at[pl.ds(sid * RPT, RPT)],
                    acc.at[pl.ds(sid * RPT, RPT)])
    pltpu.sync_copy(dst_hbm.at[wid], dstv)

    def fill(i, carry):
        ones_v[i, :] = jnp.ones((16,), jnp.float32)
        return carry

    lax.fori_loop(0, CHUNK, fill, 0)
    plsc.subcore_barrier()

    def body(j, carry):
        pltpu.sync_copy(ones_v, acc.at[dstv.at[j]], add=True)
        return carry

    lax.fori_loop(0, NCHUNKS, body, 0)
    plsc.subcore_barrier()
    pltpu.sync_copy(acc.at[pl.ds(sid * RPT, RPT)],
                    out_hbm.at[cid, pl.ds(sid * RPT, RPT)])


def _make_scatter(W, nbuf, nphase):
    pchunks = NCHUNKS // nphase
    ng = pchunks // nbuf

    @functools.partial(
        pl.kernel,
        out_type=jax.ShapeDtypeStruct((NC, N_ACC, W), jnp.float32),
        mesh=_mesh,
        compiler_params=_sc_params_nl,
        scratch_types=[
            pltpu.VMEM((pchunks, CHUNK), jnp.int32),
            pltpu.VMEM((pchunks, CHUNK), jnp.int32),
            [pltpu.VMEM((CHUNK, W), jnp.bfloat16) for _ in range(nbuf)],
            pltpu.VMEM((CHUNK, W), jnp.float32),
            pltpu.VMEM_SHARED((N_ACC, W), jnp.float32),
            [pltpu.SemaphoreType.DMA for _ in range(nbuf)],
        ],
    )
    def _scatter(xs_hbm, src_hbm, dst_hbm, zeros_hbm, out_hbm,
                 srcv, dstv, rows16, rows32, acc, sems):
        cid = lax.axis_index("c")
        sid = lax.axis_index("s")
        wid = sid * NC + cid
        pltpu.sync_copy(zeros_hbm.at[pl.ds(sid * RPT, RPT)],
                        acc.at[pl.ds(sid * RPT, RPT)])
        plsc.subcore_barrier()

        for p in range(nphase):
            pltpu.sync_copy(src_hbm.at[wid, pl.ds(p * pchunks, pchunks)],
                            srcv)
            pltpu.sync_copy(dst_hbm.at[wid, pl.ds(p * pchunks, pchunks)],
                            dstv)

            for b in range(nbuf):
                pltpu.async_copy(xs_hbm.at[srcv.at[b]], rows16[b], sems[b])

            def body(g, carry):
                jbase = g * nbuf
                for b in range(nbuf):
                    j = jbase + b
                    pltpu.make_async_copy(
                        xs_hbm.at[srcv.at[j]], rows16[b], sems[b]).wait()

                    def widen(i, c2, _rb=rows16[b]):
                        for grp in range(W // 32):
                            v = plsc.bitcast(_rb[i, pl.ds(grp * 32, 32)],
                                             jnp.int32)
                            lo = plsc.bitcast(
                                lax.shift_left(v, 16), jnp.float32)
                            hi = plsc.bitcast(
                                lax.bitwise_and(v, jnp.int32(-65536)),
                                jnp.float32)
                            rows32[i, pl.ds(grp * 32, 16)] = lo
                            rows32[i, pl.ds(grp * 32 + 16, 16)] = hi
                        return c2

                    lax.fori_loop(0, CHUNK, widen, 0)
                    nj = j + nbuf

                    @pl.when(nj < pchunks)
                    def _():
                        pltpu.async_copy(
                            xs_hbm.at[srcv.at[nj]], rows16[b], sems[b])

                    pltpu.sync_copy(rows32, acc.at[dstv.at[j]], add=True)

                return carry

            lax.fori_loop(0, ng, body, 0)

        plsc.subcore_barrier()
        pltpu.sync_copy(acc.at[pl.ds(sid * RPT, RPT)],
                        out_hbm.at[cid, pl.ds(sid * RPT, RPT)])

    return _scatter


_scatter128 = _make_scatter(128, 2, 2)
_scatter64 = _make_scatter(64, 4, 1)
_scatter32 = _make_scatter(32, 4, 1)


def _interleave(a):
    n, w = a.shape
    return a.reshape(n, w // 32, 2, 16).swapaxes(2, 3).reshape(n, w)


R = 1000
G = N // R


def _row_spec(w):
    return pl.BlockSpec((R, w), lambda i: (i, 0))


def _full_spec(shape):
    return pl.BlockSpec(shape, lambda i: tuple(0 for _ in shape))


def _tca_body(p0, p1, x, dinv_o, x1s_o, x1sb_o):
    deg = 1.0 + p0[:, 0:1] + p1[:, 0:1]
    dinv = lax.rsqrt(deg)
    dinv_o[...] = dinv
    x1s = x[...] * dinv
    x1s_o[...] = x1s
    x1sb_o[...] = x1s.astype(jnp.bfloat16)


_tc_a = pl.pallas_call(
    _tca_body,
    grid=(G,),
    in_specs=[_row_spec(DEG_W), _row_spec(DEG_W), _row_spec(128)],
    out_specs=[_row_spec(1), _row_spec(128), _row_spec(128)],
    out_shape=[
        jax.ShapeDtypeStruct((N, 1), jnp.float32),
        jax.ShapeDtypeStruct((N, 128), jnp.float32),
        jax.ShapeDtypeStruct((N, 128), jnp.bfloat16),
    ],
)


def _tcb_body(dinv, p1a, p1b, x1s, W1r, g1r, b1r, bt1r, W2r, g2r,
              x2s_o, x2sb_o):
    di = dinv[...]
    g1s = g1r[...] * _S
    W1p = W1r[...] * g1s
    b1p = b1r[...] * g1s + bt1r[...]
    psum = p1a[...].astype(jnp.float32) + p1b[...].astype(jnp.float32)
    agg = (psum + x1s[...]) * di
    h1 = jnp.maximum(
        jnp.dot(agg, W1p, preferred_element_type=jnp.float32) + b1p, 0.0)
    W2p = W2r[...] * (g2r[...] * _S)
    x2s = jnp.dot(h1, W2p, preferred_element_type=jnp.float32) * di
    x2s_o[...] = x2s
    x2sb_o[...] = x2s.astype(jnp.bfloat16)


_tc_b = pl.pallas_call(
    _tcb_body,
    grid=(G,),
    in_specs=[
        _row_spec(1), _row_spec(128), _row_spec(128), _row_spec(128),
        _full_spec((128, 128)), _full_spec((1, 128)), _full_spec((1, 128)),
        _full_spec((1, 128)), _full_spec((128, 64)), _full_spec((1, 64)),
    ],
    out_specs=[_row_spec(64), _row_spec(64)],
    out_shape=[
        jax.ShapeDtypeStruct((N, 64), jnp.float32),
        jax.ShapeDtypeStruct((N, 64), jnp.bfloat16),
    ],
)


def _tcc_body(dinv, p2a, p2b, x2s, b2r, g2r, bt2r, W3r, g3r, x3s_o, x3sb_o):
    di = dinv[...]
    g2s = g2r[...] * _S
    b2p = b2r[...] * g2s + bt2r[...]
    psum = p2a[...].astype(jnp.float32) + p2b[...].astype(jnp.float32)
    h2 = jnp.maximum((psum + x2s[...]) * di + b2p, 0.0)
    W3p = W3r[...] * (g3r[...] * _S)
    x3s = jnp.dot(h2, W3p, preferred_element_type=jnp.float32) * di
    x3s_o[...] = x3s
    x3sb_o[...] = x3s.astype(jnp.bfloat16)


_tc_c = pl.pallas_call(
    _tcc_body,
    grid=(G,),
    in_specs=[
        _row_spec(1), _row_spec(64), _row_spec(64), _row_spec(64),
        _full_spec((1, 64)), _full_spec((1, 64)), _full_spec((1, 64)),
        _full_spec((64, 32)), _full_spec((1, 32)),
    ],
    out_specs=[_row_spec(32), _row_spec(32)],
    out_shape=[
        jax.ShapeDtypeStruct((N, 32), jnp.float32),
        jax.ShapeDtypeStruct((N, 32), jnp.bfloat16),
    ],
)


def _tcd_body(dinv, p3a, p3b, x3s, b3r, g3r, bt3r, fcWr, fcbr, out_o):
    di = dinv[...]
    g3s = g3r[...] * _S
    b3p = b3r[...] * g3s + bt3r[...]
    psum = p3a[...].astype(jnp.float32) + p3b[...].astype(jnp.float32)
    h3 = jnp.maximum((psum + x3s[...]) * di + b3p, 0.0)
    out_o[...] = (jnp.dot(h3, fcWr[...], preferred_element_type=jnp.float32)
                  + fcbr[...])


_tc_d = pl.pallas_call(
    _tcd_body,
    grid=(G,),
    in_specs=[
        _row_spec(1), _row_spec(32), _row_spec(32), _row_spec(32),
        _full_spec((1, 32)), _full_spec((1, 32)), _full_spec((1, 32)),
        _full_spec((32, 1)), _full_spec((1, 1)),
    ],
    out_specs=_row_spec(1),
    out_shape=jax.ShapeDtypeStruct((N, 1), jnp.float32),
)


def kernel(x, edge_index, W1, b1, g1, bt1, W2, b2, g2, bt2,
           W3, b3, g3, bt3, fcW, fcb):
    src = edge_index[0].astype(jnp.int32)
    dst = edge_index[1].astype(jnp.int32)
    npad = E_PAD - E
    src3 = jnp.concatenate(
        [src, jnp.zeros((npad,), jnp.int32)]).reshape(NW, NCHUNKS, CHUNK)
    dst3 = jnp.concatenate(
        [dst, jnp.full((npad,), N, jnp.int32)]).reshape(NW, NCHUNKS, CHUNK)
    z16 = jnp.zeros((N_ACC, DEG_W), jnp.float32)
    z128 = jnp.zeros((N_ACC, 128), jnp.float32)
    z64 = jnp.zeros((N_ACC, 64), jnp.float32)
    z32 = jnp.zeros((N_ACC, 32), jnp.float32)

    degP = _deg_kernel(dst3, z16)
    dinv, x1s, x1sb = _tc_a(degP[0, :N], degP[1, :N], x)
    p1 = _scatter128(_interleave(x1sb), src3, dst3, z128)
    x2s, x2sb = _tc_b(dinv, p1[0, :N], p1[1, :N], x1s, W1,
                      g1.reshape(1, 128), b1.reshape(1, 128),
                      bt1.reshape(1, 128), W2, g2.reshape(1, 64))
    p2 = _scatter64(_interleave(x2sb), src3, dst3, z64)
    x3s, x3sb = _tc_c(dinv, p2[0, :N], p2[1, :N], x2s,
                      b2.reshape(1, 64), g2.reshape(1, 64), bt2.reshape(1, 64),
                      W3, g3.reshape(1, 32))
    p3 = _scatter32(_interleave(x3sb), src3, dst3, z32)
    out = _tc_d(dinv, p3[0, :N], p3[1, :N], x3s,
                b3.reshape(1, 32), g3.reshape(1, 32), bt3.reshape(1, 32),
                fcW, fcb.reshape(1, 1))
    return out
